# baseline (speedup 1.0000x reference)
import functools

import jax
import jax.numpy as jnp
from jax import lax
from jax.experimental import pallas as pl
from jax.experimental.pallas import tpu as pltpu
from jax.experimental.pallas import tpu_sc as plsc

N = 10000
E = 320000
D_IN, D_H, D_OUT = 128, 64, 128
EPS = 1e-5

NC = 2
NS = 16
TILES = NC * NS
CH = 128
CPT = 80
EPT = CPT * CH
E_PAD = TILES * EPT
RPT = 640
ACC_ROWS = NS * RPT

_mesh = plsc.VectorSubcoreMesh(core_axis_name="c", subcore_axis_name="s")


def _deg_body(dstf_hbm, out_hbm, dst_v, deg_v, red_v, sum_v, shared):
    cid = lax.axis_index("c")
    sid = lax.axis_index("s")
    wid = cid * NS + sid
    pltpu.sync_copy(dstf_hbm.at[wid], dst_v)

    zeros16 = jnp.zeros((16,), jnp.float32)
    ones16 = jnp.ones((16,), jnp.float32)

    def zero_body(i, c):
        deg_v[pl.ds(i * 16, 16)] = zeros16
        return c

    lax.fori_loop(0, ACC_ROWS // 16, zero_body, 0)

    def acc_body(i, c):
        idx = dst_v[pl.ds(i * 16, 16)]
        plsc.addupdate_scatter(deg_v, [idx], ones16)
        return c

    lax.fori_loop(0, EPT // 16, acc_body, 0)

    pltpu.sync_copy(deg_v, shared.at[sid])
    plsc.subcore_barrier()
    for t in range(NS):
        pltpu.sync_copy(shared.at[t, pl.ds(sid * RPT, RPT)], red_v.at[t])

    def red_body(v, c):
        a = red_v[0, pl.ds(v * 16, 16)]
        for t in range(1, NS):
            a = a + red_v[t, pl.ds(v * 16, 16)]
        sum_v[pl.ds(v * 16, 16)] = a
        return c

    lax.fori_loop(0, RPT // 16, red_body, 0)
    pltpu.sync_copy(sum_v, out_hbm.at[cid, pl.ds(sid * RPT, RPT)])


_deg_kernel = functools.partial(
    pl.kernel,
    out_type=jax.ShapeDtypeStruct((NC, ACC_ROWS), jnp.float32),
    mesh=_mesh,
    compiler_params=pltpu.CompilerParams(needs_layout_passes=False),
    scratch_types=[
        pltpu.VMEM((EPT,), jnp.int32),
        pltpu.VMEM((ACC_ROWS,), jnp.float32),
        pltpu.VMEM((NS, RPT), jnp.float32),
        pltpu.VMEM((RPT,), jnp.float32),
        pltpu.VMEM_SHARED((NS, ACC_ROWS), jnp.float32),
    ],
)(_deg_body)


NBUF = 4
NCHUNKS = E_PAD // CH
NCH0 = 1920
NCH1 = NCHUNKS - NCH0
CPT0 = NCH0 // NS
CPT1 = NCH1 // NS


def _scat_body(src_hbm, dst_hbm, y_hbm, out_hbm,
               src_v, dst_v, rows0, rows1, rows2, rows3,
               acc, g0, g1, g2, g3):
    cid = lax.axis_index("c")
    sid = lax.axis_index("s")
    rows = (rows0, rows1, rows2, rows3)
    gsem = (g0, g1, g2, g3)

    zeros16 = jnp.zeros((16,), jnp.float32)

    with jax.named_scope("zeroinit"):
        def zero_body(r, c):
            for k in range(D_H // 16):
                rows0[r, pl.ds(k * 16, 16)] = zeros16
            return c

        lax.fori_loop(0, CH, zero_body, 0)
        for q in range(RPT // CH):
            pltpu.sync_copy(rows0, acc.at[pl.ds(sid * RPT + q * CH, CH)])

    with jax.named_scope("idxcopy"):
        @pl.when(cid == 0)
        def _():
            base = sid * CPT0
            pltpu.sync_copy(src_hbm.at[pl.ds(base, CPT0)], src_v)
            pltpu.sync_copy(dst_hbm.at[pl.ds(base, CPT0)], dst_v)

        @pl.when(cid == 1)
        def _():
            base = NCH0 + sid * CPT1
            pltpu.sync_copy(src_hbm.at[pl.ds(base, CPT1)],
                            src_v.at[pl.ds(0, CPT1)])
            pltpu.sync_copy(dst_hbm.at[pl.ds(base, CPT1)],
                            dst_v.at[pl.ds(0, CPT1)])

    cpt = jnp.where(cid == 0, CPT0, CPT1)
    plsc.subcore_barrier()

    with jax.named_scope("mainloop"):
        for b in range(NBUF):
            pltpu.async_copy(y_hbm.at[src_v.at[b]], rows[b], gsem[b])

        def body(i, c):
            jb = i * NBUF
            for b in range(NBUF):
                pltpu.make_async_copy(y_hbm.at[src_v.at[jb + b]],
                                      rows[b], gsem[b]).wait()
                pltpu.sync_copy(rows[b], acc.at[dst_v.at[jb + b]], add=True)

                @pl.when(jb + b + NBUF < cpt)
                def _():
                    pltpu.async_copy(y_hbm.at[src_v.at[jb + b + NBUF]],
                                     rows[b], gsem[b])
            return c

        lax.fori_loop(0, cpt // NBUF, body, 0)
    plsc.subcore_barrier()
    with jax.named_scope("copyout"):
        pltpu.sync_copy(acc.at[pl.ds(sid * RPT, RPT)],
                        out_hbm.at[cid, pl.ds(sid * RPT, RPT)])


_scatter = functools.partial(
    pl.kernel,
    out_type=jax.ShapeDtypeStruct((NC, ACC_ROWS, D_H), jnp.float32),
    mesh=_mesh,
    compiler_params=pltpu.CompilerParams(use_tc_tiling_on_sc=False),
    scratch_types=[
        pltpu.VMEM((CPT0, CH), jnp.int32),
        pltpu.VMEM((CPT0, CH), jnp.int32),
        pltpu.VMEM((CH, D_H), jnp.float32),
        pltpu.VMEM((CH, D_H), jnp.float32),
        pltpu.VMEM((CH, D_H), jnp.float32),
        pltpu.VMEM((CH, D_H), jnp.float32),
        pltpu.VMEM_SHARED((ACC_ROWS, D_H), jnp.float32),
        pltpu.SemaphoreType.DMA,
        pltpu.SemaphoreType.DMA,
        pltpu.SemaphoreType.DMA,
        pltpu.SemaphoreType.DMA,
    ],
)(_scat_body)


def _tc1_body(x_ref, wr_ref, br_ref, w1_ref, degp_ref,
              res_ref, y1_ref, dis_ref):
    x = x_ref[...]
    deg = degp_ref[:N, 0:1] + degp_ref[:N, 1:2] + 1.0
    dis = lax.rsqrt(deg)
    res_ref[...] = jnp.dot(x, wr_ref[...],
                           preferred_element_type=jnp.float32) + br_ref[...]
    y1_ref[...] = jnp.dot(x, w1_ref[...],
                          preferred_element_type=jnp.float32) * dis
    dis_ref[...] = dis


def _bn(h, g, be):
    mean = jnp.mean(h, axis=0, keepdims=True)
    d = h - mean
    var = jnp.mean(d * d, axis=0, keepdims=True)
    return d * lax.rsqrt(var + EPS) * g + be


def _tc2_body(p_ref, y_ref, dis_ref, b_ref, g_ref, be_ref, res_ref, w2_ref,
              h1_ref, y2_ref):
    dis = dis_ref[...]
    s = p_ref[0, :N, :] + p_ref[1, :N, :] + y_ref[...]
    h = dis * s + b_ref[...]
    h1 = jnp.maximum(res_ref[...] + _bn(h, g_ref[...], be_ref[...]), 0.0)
    h1_ref[...] = h1
    y2_ref[...] = jnp.dot(h1, w2_ref[...],
                          preferred_element_type=jnp.float32) * dis


def _tc3_body(p_ref, y_ref, dis_ref, b_ref, g_ref, be_ref, res_ref,
              wr2_ref, br2_ref, w3_ref, res2_ref, y3a_ref, y3b_ref):
    dis = dis_ref[...]
    s = p_ref[0, :N, :] + p_ref[1, :N, :] + y_ref[...]
    h = dis * s + b_ref[...]
    h2 = jnp.maximum(res_ref[...] + _bn(h, g_ref[...], be_ref[...]), 0.0)
    res2_ref[...] = jnp.dot(h2, wr2_ref[...],
                            preferred_element_type=jnp.float32) + br2_ref[...]
    y3 = jnp.dot(h2, w3_ref[...],
                 preferred_element_type=jnp.float32) * dis
    y3a_ref[...] = y3[:, :D_H]
    y3b_ref[...] = y3[:, D_H:]


def _tc4_body(pa_ref, pb_ref, ya_ref, yb_ref, dis_ref, b_ref, g_ref, be_ref,
              res2_ref, out_ref):
    dis = dis_ref[...]
    sa = pa_ref[0, :N, :] + pa_ref[1, :N, :] + ya_ref[...]
    sb = pb_ref[0, :N, :] + pb_ref[1, :N, :] + yb_ref[...]
    s = jnp.concatenate([sa, sb], axis=1)
    h = dis * s + b_ref[...]
    z = res2_ref[...] + _bn(h, g_ref[...], be_ref[...])
    m = jnp.max(z, axis=1, keepdims=True)
    zs = z - m
    lse = jnp.log(jnp.sum(jnp.exp(zs), axis=1, keepdims=True))
    out_ref[...] = zs - lse


def _sds(shape):
    return jax.ShapeDtypeStruct(shape, jnp.float32)


_tc1 = pl.pallas_call(
    _tc1_body, out_shape=[_sds((N, D_H)), _sds((N, D_H)), _sds((N, 1))])
_tc2 = pl.pallas_call(
    _tc2_body, out_shape=[_sds((N, D_H)), _sds((N, D_H))])
_tc3 = pl.pallas_call(
    _tc3_body, out_shape=[_sds((N, D_OUT)), _sds((N, D_H)), _sds((N, D_H))])
_tc4 = pl.pallas_call(_tc4_body, out_shape=_sds((N, D_OUT)))


def kernel(x, edge_index, W1, b1, W2, b2, W3, b3,
           g1, be1, g2, be2, g3, be3, Wr, br, Wr2, br2):
    src = edge_index[0]
    dst = edge_index[1]
    pad = E_PAD - E
    src_p = jnp.concatenate(
        [src, jnp.zeros((pad,), jnp.int32)]).reshape(NCHUNKS, CH)
    dst_pf = jnp.concatenate(
        [dst, jnp.full((pad,), N, jnp.int32)]).reshape(TILES, EPT)
    dst_p = dst_pf.reshape(NCHUNKS, CH)

    degp = jnp.transpose(_deg_kernel(dst_pf))

    res, y1, dis = _tc1(x, Wr, br.reshape(1, D_H), W1, degp)

    p1 = _scatter(src_p, dst_p, y1)
    h1, y2 = _tc2(p1, y1, dis, b1.reshape(1, D_H), g1.reshape(1, D_H),
                  be1.reshape(1, D_H), res, W2)

    p2 = _scatter(src_p, dst_p, y2)
    res2, y3a, y3b = _tc3(p2, y2, dis, b2.reshape(1, D_H), g2.reshape(1, D_H),
                          be2.reshape(1, D_H), h1, Wr2,
                          br2.reshape(1, D_OUT), W3)

    p3a = _scatter(src_p, dst_p, y3a)
    p3b = _scatter(src_p, dst_p, y3b)
    out = _tc4(p3a, p3b, y3a, y3b, dis, b3.reshape(1, D_OUT),
               g3.reshape(1, D_OUT), be3.reshape(1, D_OUT), res2)
    return out


# speedup vs baseline: 3.1564x; 3.1564x over previous
import functools

import jax
import jax.numpy as jnp
from jax import lax
from jax.experimental import pallas as pl
from jax.experimental.pallas import tpu as pltpu
from jax.experimental.pallas import tpu_sc as plsc

N = 10000
E = 320000
D_IN, D_H, D_OUT = 128, 64, 128
EPS = 1e-5

NC = 2
NS = 16
TILES = NC * NS
CH = 128
CPT = 80
EPT = CPT * CH
E_PAD = TILES * EPT
RPT = 640
ACC_ROWS = NS * RPT

_mesh = plsc.VectorSubcoreMesh(core_axis_name="c", subcore_axis_name="s")


def _deg_body(dstf_hbm, out_hbm, dst_v, deg_v, red_v, sum_v, shared):
    cid = lax.axis_index("c")
    sid = lax.axis_index("s")
    wid = cid * NS + sid
    pltpu.sync_copy(dstf_hbm.at[wid], dst_v)

    zeros16 = jnp.zeros((16,), jnp.float32)
    ones16 = jnp.ones((16,), jnp.float32)

    def zero_body(i, c):
        deg_v[pl.ds(i * 16, 16)] = zeros16
        return c

    lax.fori_loop(0, ACC_ROWS // 16, zero_body, 0)

    def acc_body(i, c):
        idx = dst_v[pl.ds(i * 16, 16)]
        plsc.addupdate_scatter(deg_v, [idx], ones16)
        return c

    lax.fori_loop(0, EPT // 16, acc_body, 0)

    pltpu.sync_copy(deg_v, shared.at[sid])
    plsc.subcore_barrier()
    for t in range(NS):
        pltpu.sync_copy(shared.at[t, pl.ds(sid * RPT, RPT)], red_v.at[t])

    def red_body(v, c):
        a = red_v[0, pl.ds(v * 16, 16)]
        for t in range(1, NS):
            a = a + red_v[t, pl.ds(v * 16, 16)]
        sum_v[pl.ds(v * 16, 16)] = a
        return c

    lax.fori_loop(0, RPT // 16, red_body, 0)
    pltpu.sync_copy(sum_v, out_hbm.at[cid, pl.ds(sid * RPT, RPT)])


_deg_kernel = functools.partial(
    pl.kernel,
    out_type=jax.ShapeDtypeStruct((NC, ACC_ROWS), jnp.float32),
    mesh=_mesh,
    compiler_params=pltpu.CompilerParams(needs_layout_passes=False),
    scratch_types=[
        pltpu.VMEM((EPT,), jnp.int32),
        pltpu.VMEM((ACC_ROWS,), jnp.float32),
        pltpu.VMEM((NS, RPT), jnp.float32),
        pltpu.VMEM((RPT,), jnp.float32),
        pltpu.VMEM_SHARED((NS, ACC_ROWS), jnp.float32),
    ],
)(_deg_body)


NBUF = 4
NCHUNKS = E_PAD // CH


def _scat_body(src_hbm, dst_hbm, y_hbm, out_hbm,
               src_v, dst_v, rows0, rows1, rows2, rows3,
               acc, g0, g1, g2, g3):
    cid = lax.axis_index("c")
    sid = lax.axis_index("s")
    rows = (rows0, rows1, rows2, rows3)
    gsem = (g0, g1, g2, g3)

    zeros16 = jnp.zeros((16,), jnp.float32)

    with jax.named_scope("zeroinit"):
        def zero_body(r, c):
            for k in range(D_H // 16):
                rows0[r, pl.ds(k * 16, 16)] = zeros16
            return c

        lax.fori_loop(0, CH, zero_body, 0)
        for q in range(RPT // CH):
            pltpu.sync_copy(rows0, acc.at[pl.ds(sid * RPT + q * CH, CH)])

    with jax.named_scope("idxcopy"):
        base = (cid * NS + sid) * CPT
        pltpu.sync_copy(src_hbm.at[pl.ds(base, CPT)], src_v)
        pltpu.sync_copy(dst_hbm.at[pl.ds(base, CPT)], dst_v)

    plsc.subcore_barrier()

    with jax.named_scope("mainloop"):
        for b in range(NBUF):
            pltpu.async_copy(y_hbm.at[src_v.at[b]], rows[b], gsem[b])

        def body(i, c):
            jb = i * NBUF
            for b in range(NBUF):
                pltpu.make_async_copy(y_hbm.at[src_v.at[jb + b]],
                                      rows[b], gsem[b]).wait()
                pltpu.sync_copy(rows[b], acc.at[dst_v.at[jb + b]], add=True)

                @pl.when(jb + b + NBUF < CPT)
                def _():
                    pltpu.async_copy(y_hbm.at[src_v.at[jb + b + NBUF]],
                                     rows[b], gsem[b])
            return c

        lax.fori_loop(0, CPT // NBUF, body, 0)
    plsc.subcore_barrier()
    with jax.named_scope("copyout"):
        pltpu.sync_copy(acc.at[pl.ds(sid * RPT, RPT)],
                        out_hbm.at[cid, pl.ds(sid * RPT, RPT)])


_scatter = functools.partial(
    pl.kernel,
    out_type=jax.ShapeDtypeStruct((NC, ACC_ROWS, D_H), jnp.float32),
    mesh=_mesh,
    compiler_params=pltpu.CompilerParams(use_tc_tiling_on_sc=False),
    scratch_types=[
        pltpu.VMEM((CPT, CH), jnp.int32),
        pltpu.VMEM((CPT, CH), jnp.int32),
        pltpu.VMEM((CH, D_H), jnp.float32),
        pltpu.VMEM((CH, D_H), jnp.float32),
        pltpu.VMEM((CH, D_H), jnp.float32),
        pltpu.VMEM((CH, D_H), jnp.float32),
        pltpu.VMEM_SHARED((ACC_ROWS, D_H), jnp.float32),
        pltpu.SemaphoreType.DMA,
        pltpu.SemaphoreType.DMA,
        pltpu.SemaphoreType.DMA,
        pltpu.SemaphoreType.DMA,
    ],
)(_scat_body)


def _tc1_body(x_ref, wr_ref, br_ref, w1_ref, degp_ref,
              res_ref, y1_ref, dis_ref):
    x = x_ref[...]
    deg = degp_ref[:N, 0:1] + degp_ref[:N, 1:2] + 1.0
    dis = lax.rsqrt(deg)
    res_ref[...] = jnp.dot(x, wr_ref[...],
                           preferred_element_type=jnp.float32) + br_ref[...]
    y1_ref[...] = jnp.dot(x, w1_ref[...],
                          preferred_element_type=jnp.float32) * dis
    dis_ref[...] = dis


def _bn(h, g, be):
    mean = jnp.mean(h, axis=0, keepdims=True)
    d = h - mean
    var = jnp.mean(d * d, axis=0, keepdims=True)
    return d * lax.rsqrt(var + EPS) * g + be


def _tc2_body(p_ref, y_ref, dis_ref, b_ref, g_ref, be_ref, res_ref, w2_ref,
              h1_ref, y2_ref):
    dis = dis_ref[...]
    s = p_ref[0, :N, :] + p_ref[1, :N, :] + y_ref[...]
    h = dis * s + b_ref[...]
    h1 = jnp.maximum(res_ref[...] + _bn(h, g_ref[...], be_ref[...]), 0.0)
    h1_ref[...] = h1
    y2_ref[...] = jnp.dot(h1, w2_ref[...],
                          preferred_element_type=jnp.float32) * dis


def _tc3_body(p_ref, y_ref, dis_ref, b_ref, g_ref, be_ref, res_ref,
              wr2_ref, br2_ref, w3_ref, res2_ref, y3a_ref, y3b_ref):
    dis = dis_ref[...]
    s = p_ref[0, :N, :] + p_ref[1, :N, :] + y_ref[...]
    h = dis * s + b_ref[...]
    h2 = jnp.maximum(res_ref[...] + _bn(h, g_ref[...], be_ref[...]), 0.0)
    res2_ref[...] = jnp.dot(h2, wr2_ref[...],
                            preferred_element_type=jnp.float32) + br2_ref[...]
    y3 = jnp.dot(h2, w3_ref[...],
                 preferred_element_type=jnp.float32) * dis
    y3a_ref[...] = y3[:, :D_H]
    y3b_ref[...] = y3[:, D_H:]


def _tc4_body(pa_ref, pb_ref, ya_ref, yb_ref, dis_ref, b_ref, g_ref, be_ref,
              res2_ref, out_ref):
    dis = dis_ref[...]
    sa = pa_ref[0, :N, :] + pa_ref[1, :N, :] + ya_ref[...]
    sb = pb_ref[0, :N, :] + pb_ref[1, :N, :] + yb_ref[...]
    s = jnp.concatenate([sa, sb], axis=1)
    h = dis * s + b_ref[...]
    z = res2_ref[...] + _bn(h, g_ref[...], be_ref[...])
    m = jnp.max(z, axis=1, keepdims=True)
    zs = z - m
    lse = jnp.log(jnp.sum(jnp.exp(zs), axis=1, keepdims=True))
    out_ref[...] = zs - lse


def _sds(shape):
    return jax.ShapeDtypeStruct(shape, jnp.float32)


_tc1 = pl.pallas_call(
    _tc1_body, out_shape=[_sds((N, D_H)), _sds((N, D_H)), _sds((N, 1))])
_tc2 = pl.pallas_call(
    _tc2_body, out_shape=[_sds((N, D_H)), _sds((N, D_H))])
_tc3 = pl.pallas_call(
    _tc3_body, out_shape=[_sds((N, D_OUT)), _sds((N, D_H)), _sds((N, D_H))])
_tc4 = pl.pallas_call(_tc4_body, out_shape=_sds((N, D_OUT)))


def kernel(x, edge_index, W1, b1, W2, b2, W3, b3,
           g1, be1, g2, be2, g3, be3, Wr, br, Wr2, br2):
    src = edge_index[0]
    dst = edge_index[1]
    pad = E_PAD - E
    pad_src = jnp.arange(pad, dtype=jnp.int32) % N
    pad_dst = N + jnp.arange(pad, dtype=jnp.int32) % (ACC_ROWS - N)
    src_p = jnp.concatenate([src, pad_src]).reshape(NCHUNKS, CH)
    dst_pf = jnp.concatenate([dst, pad_dst]).reshape(TILES, EPT)
    dst_p = dst_pf.reshape(NCHUNKS, CH)

    degp = jnp.transpose(_deg_kernel(dst_pf))

    res, y1, dis = _tc1(x, Wr, br.reshape(1, D_H), W1, degp)

    p1 = _scatter(src_p, dst_p, y1)
    h1, y2 = _tc2(p1, y1, dis, b1.reshape(1, D_H), g1.reshape(1, D_H),
                  be1.reshape(1, D_H), res, W2)

    p2 = _scatter(src_p, dst_p, y2)
    res2, y3a, y3b = _tc3(p2, y2, dis, b2.reshape(1, D_H), g2.reshape(1, D_H),
                          be2.reshape(1, D_H), h1, Wr2,
                          br2.reshape(1, D_OUT), W3)

    p3a = _scatter(src_p, dst_p, y3a)
    p3b = _scatter(src_p, dst_p, y3b)
    out = _tc4(p3a, p3b, y3a, y3b, dis, b3.reshape(1, D_OUT),
               g3.reshape(1, D_OUT), be3.reshape(1, D_OUT), res2)
    return out


# speedup vs baseline: 3.4779x; 1.1018x over previous
import functools

import jax
import jax.numpy as jnp
from jax import lax
from jax.experimental import pallas as pl
from jax.experimental.pallas import tpu as pltpu
from jax.experimental.pallas import tpu_sc as plsc

N = 10000
E = 320000
D_IN, D_H, D_OUT = 128, 64, 128
EPS = 1e-5

NC = 2
NS = 16
TILES = NC * NS
CH = 128
CPT = 80
EPT = CPT * CH
E_PAD = TILES * EPT
RPT = 640
ACC_ROWS = NS * RPT

_mesh = plsc.VectorSubcoreMesh(core_axis_name="c", subcore_axis_name="s")


def _deg_body(dstf_hbm, out_hbm, dst_v, deg_v, red_v, sum_v, shared):
    cid = lax.axis_index("c")
    sid = lax.axis_index("s")
    wid = cid * NS + sid
    pltpu.sync_copy(dstf_hbm.at[wid], dst_v)

    zeros16 = jnp.zeros((16,), jnp.float32)
    ones16 = jnp.ones((16,), jnp.float32)

    def zero_body(i, c):
        deg_v[pl.ds(i * 16, 16)] = zeros16
        return c

    lax.fori_loop(0, ACC_ROWS // 16, zero_body, 0)

    def acc_body(i, c):
        idx = dst_v[pl.ds(i * 16, 16)]
        plsc.addupdate_scatter(deg_v, [idx], ones16)
        return c

    lax.fori_loop(0, EPT // 16, acc_body, 0)

    pltpu.sync_copy(deg_v, shared.at[sid])
    plsc.subcore_barrier()
    for t in range(NS):
        pltpu.sync_copy(shared.at[t, pl.ds(sid * RPT, RPT)], red_v.at[t])

    def red_body(v, c):
        a = red_v[0, pl.ds(v * 16, 16)]
        for t in range(1, NS):
            a = a + red_v[t, pl.ds(v * 16, 16)]
        sum_v[pl.ds(v * 16, 16)] = a
        return c

    lax.fori_loop(0, RPT // 16, red_body, 0)
    pltpu.sync_copy(sum_v, out_hbm.at[cid, pl.ds(sid * RPT, RPT)])


_deg_kernel = functools.partial(
    pl.kernel,
    out_type=jax.ShapeDtypeStruct((NC, ACC_ROWS), jnp.float32),
    mesh=_mesh,
    compiler_params=pltpu.CompilerParams(needs_layout_passes=False),
    scratch_types=[
        pltpu.VMEM((EPT,), jnp.int32),
        pltpu.VMEM((ACC_ROWS,), jnp.float32),
        pltpu.VMEM((NS, RPT), jnp.float32),
        pltpu.VMEM((RPT,), jnp.float32),
        pltpu.VMEM_SHARED((NS, ACC_ROWS), jnp.float32),
    ],
)(_deg_body)


NBUF = 4
NCHUNKS = E_PAD // CH


def _scat_body(src_hbm, dst_hbm, y_hbm, out_hbm,
               src_v, dst_v, rows0, rows1, rows2, rows3,
               acc, g0, g1, g2, g3):
    cid = lax.axis_index("c")
    sid = lax.axis_index("s")
    rows = (rows0, rows1, rows2, rows3)
    gsem = (g0, g1, g2, g3)

    zeros16 = jnp.zeros((16,), jnp.float32)

    with jax.named_scope("zeroinit"):
        def zero_body(r, c):
            for k in range(D_H // 16):
                rows0[r, pl.ds(k * 16, 16)] = zeros16
            return c

        lax.fori_loop(0, CH, zero_body, 0)
        for q in range(RPT // CH):
            pltpu.sync_copy(rows0, acc.at[pl.ds(sid * RPT + q * CH, CH)])

    with jax.named_scope("idxcopy"):
        base = (cid * NS + sid) * CPT
        pltpu.sync_copy(src_hbm.at[pl.ds(base, CPT)], src_v)
        pltpu.sync_copy(dst_hbm.at[pl.ds(base, CPT)], dst_v)

    plsc.subcore_barrier()

    with jax.named_scope("mainloop"):
        for b in range(NBUF):
            pltpu.async_copy(y_hbm.at[src_v.at[b]], rows[b], gsem[b])

        def body(i, c):
            jb = i * NBUF
            for b in range(NBUF):
                pltpu.make_async_copy(y_hbm.at[src_v.at[jb + b]],
                                      rows[b], gsem[b]).wait()
                pltpu.sync_copy(rows[b], acc.at[dst_v.at[jb + b]], add=True)

                @pl.when(jb + b + NBUF < CPT)
                def _():
                    pltpu.async_copy(y_hbm.at[src_v.at[jb + b + NBUF]],
                                     rows[b], gsem[b])
            return c

        lax.fori_loop(0, CPT // NBUF, body, 0)
    plsc.subcore_barrier()
    with jax.named_scope("copyout"):
        pltpu.sync_copy(acc.at[pl.ds(sid * RPT, RPT)],
                        out_hbm.at[cid, pl.ds(sid * RPT, RPT)])


_scatter = functools.partial(
    pl.kernel,
    out_type=jax.ShapeDtypeStruct((NC, ACC_ROWS, D_H), jnp.float32),
    mesh=_mesh,
    compiler_params=pltpu.CompilerParams(use_tc_tiling_on_sc=False),
    scratch_types=[
        pltpu.VMEM((CPT, CH), jnp.int32),
        pltpu.VMEM((CPT, CH), jnp.int32),
        pltpu.VMEM((CH, D_H), jnp.float32),
        pltpu.VMEM((CH, D_H), jnp.float32),
        pltpu.VMEM((CH, D_H), jnp.float32),
        pltpu.VMEM((CH, D_H), jnp.float32),
        pltpu.VMEM_SHARED((ACC_ROWS, D_H), jnp.float32),
        pltpu.SemaphoreType.DMA,
        pltpu.SemaphoreType.DMA,
        pltpu.SemaphoreType.DMA,
        pltpu.SemaphoreType.DMA,
    ],
)(_scat_body)


CPT3 = NCHUNKS // NS


def _scat3_body(srca_hbm, srcb_hbm, dst_hbm, y_hbm, out_hbm,
                src_v, dst_v, rows0, rows1, rows2, rows3,
                acc, g0, g1, g2, g3):
    cid = lax.axis_index("c")
    sid = lax.axis_index("s")
    rows = (rows0, rows1, rows2, rows3)
    gsem = (g0, g1, g2, g3)

    zeros16 = jnp.zeros((16,), jnp.float32)

    def zero_body(r, c):
        for k in range(D_H // 16):
            rows0[r, pl.ds(k * 16, 16)] = zeros16
        return c

    lax.fori_loop(0, CH, zero_body, 0)
    for q in range(RPT // CH):
        pltpu.sync_copy(rows0, acc.at[pl.ds(sid * RPT + q * CH, CH)])

    base = sid * CPT3

    @pl.when(cid == 0)
    def _():
        pltpu.sync_copy(srca_hbm.at[pl.ds(base, CPT3)], src_v)

    @pl.when(cid == 1)
    def _():
        pltpu.sync_copy(srcb_hbm.at[pl.ds(base, CPT3)], src_v)

    pltpu.sync_copy(dst_hbm.at[pl.ds(base, CPT3)], dst_v)
    plsc.subcore_barrier()

    for b in range(NBUF):
        pltpu.async_copy(y_hbm.at[src_v.at[b]], rows[b], gsem[b])

    def body(i, c):
        jb = i * NBUF
        for b in range(NBUF):
            pltpu.make_async_copy(y_hbm.at[src_v.at[jb + b]],
                                  rows[b], gsem[b]).wait()
            pltpu.sync_copy(rows[b], acc.at[dst_v.at[jb + b]], add=True)

            @pl.when(jb + b + NBUF < CPT3)
            def _():
                pltpu.async_copy(y_hbm.at[src_v.at[jb + b + NBUF]],
                                 rows[b], gsem[b])
        return c

    lax.fori_loop(0, CPT3 // NBUF, body, 0)
    plsc.subcore_barrier()
    pltpu.sync_copy(acc.at[pl.ds(sid * RPT, RPT)],
                    out_hbm.at[cid, pl.ds(sid * RPT, RPT)])


_scatter3 = functools.partial(
    pl.kernel,
    out_type=jax.ShapeDtypeStruct((NC, ACC_ROWS, D_H), jnp.float32),
    mesh=_mesh,
    compiler_params=pltpu.CompilerParams(use_tc_tiling_on_sc=False),
    scratch_types=[
        pltpu.VMEM((CPT3, CH), jnp.int32),
        pltpu.VMEM((CPT3, CH), jnp.int32),
        pltpu.VMEM((CH, D_H), jnp.float32),
        pltpu.VMEM((CH, D_H), jnp.float32),
        pltpu.VMEM((CH, D_H), jnp.float32),
        pltpu.VMEM((CH, D_H), jnp.float32),
        pltpu.VMEM_SHARED((ACC_ROWS, D_H), jnp.float32),
        pltpu.SemaphoreType.DMA,
        pltpu.SemaphoreType.DMA,
        pltpu.SemaphoreType.DMA,
        pltpu.SemaphoreType.DMA,
    ],
)(_scat3_body)


def _tc1_body(x_ref, wr_ref, br_ref, w1_ref, degp_ref,
              res_ref, y1_ref, dis_ref):
    x = x_ref[...]
    deg = degp_ref[:N, 0:1] + degp_ref[:N, 1:2] + 1.0
    dis = lax.rsqrt(deg)
    res_ref[...] = jnp.dot(x, wr_ref[...],
                           preferred_element_type=jnp.float32) + br_ref[...]
    y1 = jnp.dot(x, w1_ref[...], preferred_element_type=jnp.float32) * dis
    y1_ref[:, :D_H] = y1
    y1_ref[:, D_H:] = y1
    dis_ref[...] = dis


def _bn(h, g, be):
    mean = jnp.mean(h, axis=0, keepdims=True)
    d = h - mean
    var = jnp.mean(d * d, axis=0, keepdims=True)
    return d * lax.rsqrt(var + EPS) * g + be


def _tc2_body(p_ref, y_ref, dis_ref, b_ref, g_ref, be_ref, res_ref, w2_ref,
              h1_ref, y2_ref):
    dis = dis_ref[...]
    s = p_ref[0, :N, :] + p_ref[1, :N, :] + y_ref[:, :D_H]
    h = dis * s + b_ref[...]
    h1 = jnp.maximum(res_ref[...] + _bn(h, g_ref[...], be_ref[...]), 0.0)
    h1_ref[...] = h1
    y2 = jnp.dot(h1, w2_ref[...], preferred_element_type=jnp.float32) * dis
    y2_ref[:, :D_H] = y2
    y2_ref[:, D_H:] = y2


def _tc3_body(p_ref, y_ref, dis_ref, b_ref, g_ref, be_ref, res_ref,
              wr2_ref, br2_ref, w3_ref, res2_ref, y3_ref):
    dis = dis_ref[...]
    s = p_ref[0, :N, :] + p_ref[1, :N, :] + y_ref[:, :D_H]
    h = dis * s + b_ref[...]
    h2 = jnp.maximum(res_ref[...] + _bn(h, g_ref[...], be_ref[...]), 0.0)
    res2_ref[...] = jnp.dot(h2, wr2_ref[...],
                            preferred_element_type=jnp.float32) + br2_ref[...]
    y3_ref[...] = jnp.dot(h2, w3_ref[...],
                          preferred_element_type=jnp.float32) * dis


def _tc4_body(p_ref, y_ref, dis_ref, b_ref, g_ref, be_ref,
              res2_ref, out_ref):
    dis = dis_ref[...]
    sa = p_ref[0, :N, :] + y_ref[:, :D_H]
    sb = p_ref[1, :N, :] + y_ref[:, D_H:]
    s = jnp.concatenate([sa, sb], axis=1)
    h = dis * s + b_ref[...]
    z = res2_ref[...] + _bn(h, g_ref[...], be_ref[...])
    m = jnp.max(z, axis=1, keepdims=True)
    zs = z - m
    lse = jnp.log(jnp.sum(jnp.exp(zs), axis=1, keepdims=True))
    out_ref[...] = zs - lse


def _sds(shape):
    return jax.ShapeDtypeStruct(shape, jnp.float32)


_tc1 = pl.pallas_call(
    _tc1_body, out_shape=[_sds((N, D_H)), _sds((N, 2 * D_H)), _sds((N, 1))])
_tc2 = pl.pallas_call(
    _tc2_body, out_shape=[_sds((N, D_H)), _sds((N, 2 * D_H))])
_tc3 = pl.pallas_call(
    _tc3_body, out_shape=[_sds((N, D_OUT)), _sds((N, D_OUT))])
_tc4 = pl.pallas_call(_tc4_body, out_shape=_sds((N, D_OUT)))


def kernel(x, edge_index, W1, b1, W2, b2, W3, b3,
           g1, be1, g2, be2, g3, be3, Wr, br, Wr2, br2):
    src = edge_index[0]
    dst = edge_index[1]
    pad = E_PAD - E
    pad_src = jnp.arange(pad, dtype=jnp.int32) % N
    pad_dst = N + jnp.arange(pad, dtype=jnp.int32) % (ACC_ROWS - N)
    src_f = jnp.concatenate([src, pad_src])
    srca_p = (2 * src_f).reshape(NCHUNKS, CH)
    srcb_p = (2 * src_f + 1).reshape(NCHUNKS, CH)
    dst_pf = jnp.concatenate([dst, pad_dst]).reshape(TILES, EPT)
    dst_p = dst_pf.reshape(NCHUNKS, CH)

    degp = jnp.transpose(_deg_kernel(dst_pf))

    res, y1, dis = _tc1(x, Wr, br.reshape(1, D_H), W1, degp)

    p1 = _scatter(srca_p, dst_p, y1.reshape(2 * N, D_H))
    h1, y2 = _tc2(p1, y1, dis, b1.reshape(1, D_H), g1.reshape(1, D_H),
                  be1.reshape(1, D_H), res, W2)

    p2 = _scatter(srca_p, dst_p, y2.reshape(2 * N, D_H))
    res2, y3 = _tc3(p2, y2, dis, b2.reshape(1, D_H), g2.reshape(1, D_H),
                    be2.reshape(1, D_H), h1, Wr2,
                    br2.reshape(1, D_OUT), W3)

    p3 = _scatter3(srca_p, srcb_p, dst_p, y3.reshape(2 * N, D_H))
    out = _tc4(p3, y3, dis, b3.reshape(1, D_OUT),
               g3.reshape(1, D_OUT), be3.reshape(1, D_OUT), res2)
    return out


# speedup vs baseline: 3.7866x; 1.0888x over previous
import functools

import jax
import jax.numpy as jnp
from jax import lax
from jax.experimental import pallas as pl
from jax.experimental.pallas import tpu as pltpu
from jax.experimental.pallas import tpu_sc as plsc

N = 10000
E = 320000
D_IN, D_H, D_OUT = 128, 64, 128
EPS = 1e-5

NC = 2
NS = 16
TILES = NC * NS
CH = 128
CPT = 80
EPT = CPT * CH
E_PAD = TILES * EPT
RPT = 640
ACC_ROWS = NS * RPT

_mesh = plsc.VectorSubcoreMesh(core_axis_name="c", subcore_axis_name="s")


def _deg_body(dstf_hbm, out_hbm, dst_v, deg_v, red_v, sum_v, shared):
    cid = lax.axis_index("c")
    sid = lax.axis_index("s")
    wid = cid * NS + sid
    pltpu.sync_copy(dstf_hbm.at[wid], dst_v)

    zeros16 = jnp.zeros((16,), jnp.float32)
    ones16 = jnp.ones((16,), jnp.float32)

    def zero_body(i, c):
        deg_v[pl.ds(i * 16, 16)] = zeros16
        return c

    lax.fori_loop(0, ACC_ROWS // 16, zero_body, 0)

    def acc_body(i, c):
        idx = dst_v[pl.ds(i * 16, 16)]
        plsc.addupdate_scatter(deg_v, [idx], ones16)
        return c

    lax.fori_loop(0, EPT // 16, acc_body, 0)

    pltpu.sync_copy(deg_v, shared.at[sid])
    plsc.subcore_barrier()
    for t in range(NS):
        pltpu.sync_copy(shared.at[t, pl.ds(sid * RPT, RPT)], red_v.at[t])

    def red_body(v, c):
        a = red_v[0, pl.ds(v * 16, 16)]
        for t in range(1, NS):
            a = a + red_v[t, pl.ds(v * 16, 16)]
        sum_v[pl.ds(v * 16, 16)] = a
        return c

    lax.fori_loop(0, RPT // 16, red_body, 0)
    pltpu.sync_copy(sum_v, out_hbm.at[cid, pl.ds(sid * RPT, RPT)])


_deg_kernel = functools.partial(
    pl.kernel,
    out_type=jax.ShapeDtypeStruct((NC, ACC_ROWS), jnp.float32),
    mesh=_mesh,
    compiler_params=pltpu.CompilerParams(needs_layout_passes=False),
    scratch_types=[
        pltpu.VMEM((EPT,), jnp.int32),
        pltpu.VMEM((ACC_ROWS,), jnp.float32),
        pltpu.VMEM((NS, RPT), jnp.float32),
        pltpu.VMEM((RPT,), jnp.float32),
        pltpu.VMEM_SHARED((NS, ACC_ROWS), jnp.float32),
    ],
)(_deg_body)


NBUF = 4
NCHUNKS = E_PAD // CH


def _scat_body(src_hbm, dst_hbm, y_hbm, out_hbm,
               src_v, dst_v, rows0, rows1, rows2, rows3,
               acc, g0, g1, g2, g3):
    cid = lax.axis_index("c")
    sid = lax.axis_index("s")
    rows = (rows0, rows1, rows2, rows3)
    gsem = (g0, g1, g2, g3)

    zeros16 = jnp.zeros((16,), jnp.float32)

    with jax.named_scope("zeroinit"):
        def zero_body(r, c):
            for k in range(D_H // 16):
                rows0[r, pl.ds(k * 16, 16)] = zeros16
            return c

        lax.fori_loop(0, CH, zero_body, 0)
        for q in range(RPT // CH):
            pltpu.sync_copy(rows0, acc.at[pl.ds(sid * RPT + q * CH, CH)])

    with jax.named_scope("idxcopy"):
        base = (cid * NS + sid) * CPT
        pltpu.sync_copy(src_hbm.at[pl.ds(base, CPT)], src_v)
        pltpu.sync_copy(dst_hbm.at[pl.ds(base, CPT)], dst_v)

    plsc.subcore_barrier()

    with jax.named_scope("mainloop"):
        for b in range(NBUF):
            pltpu.async_copy(y_hbm.at[src_v.at[b]], rows[b], gsem[b])

        def body(i, c):
            jb = i * NBUF
            for b in range(NBUF):
                pltpu.make_async_copy(y_hbm.at[src_v.at[jb + b]],
                                      rows[b], gsem[b]).wait()
                pltpu.sync_copy(rows[b], acc.at[dst_v.at[jb + b]], add=True)

                @pl.when(jb + b + NBUF < CPT)
                def _():
                    pltpu.async_copy(y_hbm.at[src_v.at[jb + b + NBUF]],
                                     rows[b], gsem[b])
            return c

        lax.fori_loop(0, CPT // NBUF, body, 0)
    plsc.subcore_barrier()
    with jax.named_scope("copyout"):
        pltpu.sync_copy(acc.at[pl.ds(sid * RPT, RPT)],
                        out_hbm.at[cid, pl.ds(sid * RPT, RPT), pl.ds(0, D_H)])


_scatter = functools.partial(
    pl.kernel,
    out_type=jax.ShapeDtypeStruct((NC, ACC_ROWS, 2 * D_H), jnp.float32),
    mesh=_mesh,
    compiler_params=pltpu.CompilerParams(use_tc_tiling_on_sc=False),
    scratch_types=[
        pltpu.VMEM((CPT, CH), jnp.int32),
        pltpu.VMEM((CPT, CH), jnp.int32),
        pltpu.VMEM((CH, D_H), jnp.float32),
        pltpu.VMEM((CH, D_H), jnp.float32),
        pltpu.VMEM((CH, D_H), jnp.float32),
        pltpu.VMEM((CH, D_H), jnp.float32),
        pltpu.VMEM_SHARED((ACC_ROWS, D_H), jnp.float32),
        pltpu.SemaphoreType.DMA,
        pltpu.SemaphoreType.DMA,
        pltpu.SemaphoreType.DMA,
        pltpu.SemaphoreType.DMA,
    ],
)(_scat_body)


CPT3 = NCHUNKS // NS


def _scat3_body(srca_hbm, srcb_hbm, dst_hbm, y_hbm, out_hbm,
                src_v, dst_v, rows0, rows1, rows2, rows3,
                acc, g0, g1, g2, g3):
    cid = lax.axis_index("c")
    sid = lax.axis_index("s")
    rows = (rows0, rows1, rows2, rows3)
    gsem = (g0, g1, g2, g3)

    zeros16 = jnp.zeros((16,), jnp.float32)

    def zero_body(r, c):
        for k in range(D_H // 16):
            rows0[r, pl.ds(k * 16, 16)] = zeros16
        return c

    lax.fori_loop(0, CH, zero_body, 0)
    for q in range(RPT // CH):
        pltpu.sync_copy(rows0, acc.at[pl.ds(sid * RPT + q * CH, CH)])

    base = sid * CPT3

    @pl.when(cid == 0)
    def _():
        pltpu.sync_copy(srca_hbm.at[pl.ds(base, CPT3)], src_v)

    @pl.when(cid == 1)
    def _():
        pltpu.sync_copy(srcb_hbm.at[pl.ds(base, CPT3)], src_v)

    pltpu.sync_copy(dst_hbm.at[pl.ds(base, CPT3)], dst_v)
    plsc.subcore_barrier()

    for b in range(NBUF):
        pltpu.async_copy(y_hbm.at[src_v.at[b]], rows[b], gsem[b])

    def body(i, c):
        jb = i * NBUF
        for b in range(NBUF):
            pltpu.make_async_copy(y_hbm.at[src_v.at[jb + b]],
                                  rows[b], gsem[b]).wait()
            pltpu.sync_copy(rows[b], acc.at[dst_v.at[jb + b]], add=True)

            @pl.when(jb + b + NBUF < CPT3)
            def _():
                pltpu.async_copy(y_hbm.at[src_v.at[jb + b + NBUF]],
                                 rows[b], gsem[b])
        return c

    lax.fori_loop(0, CPT3 // NBUF, body, 0)
    plsc.subcore_barrier()
    pltpu.sync_copy(acc.at[pl.ds(sid * RPT, RPT)],
                    out_hbm.at[cid, pl.ds(sid * RPT, RPT), pl.ds(0, D_H)])


_scatter3 = functools.partial(
    pl.kernel,
    out_type=jax.ShapeDtypeStruct((NC, ACC_ROWS, 2 * D_H), jnp.float32),
    mesh=_mesh,
    compiler_params=pltpu.CompilerParams(use_tc_tiling_on_sc=False),
    scratch_types=[
        pltpu.VMEM((CPT3, CH), jnp.int32),
        pltpu.VMEM((CPT3, CH), jnp.int32),
        pltpu.VMEM((CH, D_H), jnp.float32),
        pltpu.VMEM((CH, D_H), jnp.float32),
        pltpu.VMEM((CH, D_H), jnp.float32),
        pltpu.VMEM((CH, D_H), jnp.float32),
        pltpu.VMEM_SHARED((ACC_ROWS, D_H), jnp.float32),
        pltpu.SemaphoreType.DMA,
        pltpu.SemaphoreType.DMA,
        pltpu.SemaphoreType.DMA,
        pltpu.SemaphoreType.DMA,
    ],
)(_scat3_body)


def _tc1_body(x_ref, wr_ref, br_ref, w1_ref, degp_ref,
              res_ref, y1_ref, dis_ref):
    x = x_ref[...]
    deg = degp_ref[:N, 0:1] + degp_ref[:N, 1:2] + 1.0
    dis = lax.rsqrt(deg)
    res_ref[...] = jnp.dot(x, wr_ref[...],
                           preferred_element_type=jnp.float32) + br_ref[...]
    y1 = jnp.dot(x, w1_ref[...], preferred_element_type=jnp.float32) * dis
    y1_ref[:, :D_H] = y1
    y1_ref[:, D_H:] = y1
    dis_ref[...] = dis


def _bn(h, g, be):
    mean = jnp.mean(h, axis=0, keepdims=True)
    d = h - mean
    var = jnp.mean(d * d, axis=0, keepdims=True)
    return d * lax.rsqrt(var + EPS) * g + be


def _tc2_body(p_ref, y_ref, dis_ref, b_ref, g_ref, be_ref, res_ref, w2_ref,
              h1_ref, y2_ref):
    dis = dis_ref[...]
    s = p_ref[0, :N, :D_H] + p_ref[1, :N, :D_H] + y_ref[:, :D_H]
    h = dis * s + b_ref[...]
    h1 = jnp.maximum(res_ref[...] + _bn(h, g_ref[...], be_ref[...]), 0.0)
    h1_ref[...] = h1
    y2 = jnp.dot(h1, w2_ref[...], preferred_element_type=jnp.float32) * dis
    y2_ref[:, :D_H] = y2
    y2_ref[:, D_H:] = y2


def _tc3_body(p_ref, y_ref, dis_ref, b_ref, g_ref, be_ref, res_ref,
              wr2_ref, br2_ref, w3_ref, res2_ref, y3_ref):
    dis = dis_ref[...]
    s = p_ref[0, :N, :D_H] + p_ref[1, :N, :D_H] + y_ref[:, :D_H]
    h = dis * s + b_ref[...]
    h2 = jnp.maximum(res_ref[...] + _bn(h, g_ref[...], be_ref[...]), 0.0)
    res2_ref[...] = jnp.dot(h2, wr2_ref[...],
                            preferred_element_type=jnp.float32) + br2_ref[...]
    y3_ref[...] = jnp.dot(h2, w3_ref[...],
                          preferred_element_type=jnp.float32) * dis


def _tc4_body(p_ref, y_ref, dis_ref, b_ref, g_ref, be_ref,
              res2_ref, out_ref):
    dis = dis_ref[...]
    sa = p_ref[0, :N, :D_H] + y_ref[:, :D_H]
    sb = p_ref[1, :N, :D_H] + y_ref[:, D_H:]
    s = jnp.concatenate([sa, sb], axis=1)
    h = dis * s + b_ref[...]
    z = res2_ref[...] + _bn(h, g_ref[...], be_ref[...])
    m = jnp.max(z, axis=1, keepdims=True)
    zs = z - m
    lse = jnp.log(jnp.sum(jnp.exp(zs), axis=1, keepdims=True))
    out_ref[...] = zs - lse


def _sds(shape):
    return jax.ShapeDtypeStruct(shape, jnp.float32)


_tc1 = pl.pallas_call(
    _tc1_body, out_shape=[_sds((N, D_H)), _sds((N, 2 * D_H)), _sds((N, 1))])
_tc2 = pl.pallas_call(
    _tc2_body, out_shape=[_sds((N, D_H)), _sds((N, 2 * D_H))])
_tc3 = pl.pallas_call(
    _tc3_body, out_shape=[_sds((N, D_OUT)), _sds((N, D_OUT))])
_tc4 = pl.pallas_call(_tc4_body, out_shape=_sds((N, D_OUT)))


def kernel(x, edge_index, W1, b1, W2, b2, W3, b3,
           g1, be1, g2, be2, g3, be3, Wr, br, Wr2, br2):
    src = edge_index[0]
    dst = edge_index[1]
    pad = E_PAD - E
    pad_src = jnp.arange(pad, dtype=jnp.int32) % N
    pad_dst = N + jnp.arange(pad, dtype=jnp.int32) % (ACC_ROWS - N)
    src_f = jnp.concatenate([src, pad_src])
    srca_p = (2 * src_f).reshape(NCHUNKS, CH)
    srcb_p = (2 * src_f + 1).reshape(NCHUNKS, CH)
    dst_pf = jnp.concatenate([dst, pad_dst]).reshape(TILES, EPT)
    dst_p = dst_pf.reshape(NCHUNKS, CH)

    degp = jnp.transpose(_deg_kernel(dst_pf))

    res, y1, dis = _tc1(x, Wr, br.reshape(1, D_H), W1, degp)

    p1 = _scatter(srca_p, dst_p, y1.reshape(2 * N, D_H))
    h1, y2 = _tc2(p1, y1, dis, b1.reshape(1, D_H), g1.reshape(1, D_H),
                  be1.reshape(1, D_H), res, W2)

    p2 = _scatter(srca_p, dst_p, y2.reshape(2 * N, D_H))
    res2, y3 = _tc3(p2, y2, dis, b2.reshape(1, D_H), g2.reshape(1, D_H),
                    be2.reshape(1, D_H), h1, Wr2,
                    br2.reshape(1, D_OUT), W3)

    p3 = _scatter3(srca_p, srcb_p, dst_p, y3.reshape(2 * N, D_H))
    out = _tc4(p3, y3, dis, b3.reshape(1, D_OUT),
               g3.reshape(1, D_OUT), be3.reshape(1, D_OUT), res2)
    return out
